# baseline (device time: 27843 ns/iter reference)
import jax
import jax.numpy as jnp
from jax import lax
from jax.experimental import pallas as pl
from jax.experimental.pallas import tpu as pltpu

N_Z = 2


def kernel(x, router, W1, W2):
    t_per, d = x.shape
    e_per = W1.shape[0]
    router_t = router.T

    def body(x_ref, rt_ref, W1_ref, W2_ref, out_ref,
             xg_ref, rg_ref, pbuf_ref, cbuf_ref, send_sems, recv_sems):
        my_x = lax.axis_index("x")
        my_y = lax.axis_index("y")
        my_z = lax.axis_index("z")
        other_z = 1 - my_z
        partner = (my_x, my_y, other_z)

        barrier_sem = pltpu.get_barrier_semaphore()
        pl.semaphore_signal(barrier_sem, inc=1, device_id=partner,
                            device_id_type=pl.DeviceIdType.MESH)
        pl.semaphore_wait(barrier_sem, 1)

        rdma_x = pltpu.make_async_remote_copy(
            src_ref=x_ref, dst_ref=xg_ref.at[my_z],
            send_sem=send_sems.at[0], recv_sem=recv_sems.at[0],
            device_id=partner, device_id_type=pl.DeviceIdType.MESH)
        rdma_r = pltpu.make_async_remote_copy(
            src_ref=rt_ref, dst_ref=rg_ref.at[my_z],
            send_sem=send_sems.at[1], recv_sem=recv_sems.at[1],
            device_id=partner, device_id_type=pl.DeviceIdType.MESH)
        rdma_x.start()
        rdma_r.start()
        xg_ref[my_z] = x_ref[...]
        rg_ref[my_z] = rt_ref[...]
        rdma_x.wait()
        rdma_r.wait()

        for s in range(N_Z):
            xs = xg_ref[s]
            g0 = lax.dot_general(xs, rg_ref[0], (((1,), (1,)), ((), ())),
                                 preferred_element_type=jnp.float32)
            g1 = lax.dot_general(xs, rg_ref[1], (((1,), (1,)), ((), ())),
                                 preferred_element_type=jnp.float32)
            a0, a1 = g0[:, 0:1], g0[:, 1:2]
            b0, b1 = g1[:, 0:1], g1[:, 1:2]
            ma, sa = jnp.maximum(a0, a1), jnp.minimum(a0, a1)
            mb, sb = jnp.maximum(b0, b1), jnp.minimum(b0, b1)
            m1 = jnp.maximum(ma, mb)
            m2 = jnp.where(ma >= mb, jnp.maximum(sa, mb), jnp.maximum(sb, ma))
            t1 = 1.0 / (1.0 + jnp.exp(m2 - m1))
            t2 = 1.0 - t1

            def wexp(g):
                return jnp.where(g == m1, t1, jnp.where(g == m2, t2, 0.0))

            gm0 = jnp.where(my_z == 0, a0, b0)
            gm1 = jnp.where(my_z == 0, a1, b1)
            w0 = wexp(gm0)
            w1 = wexp(gm1)

            h0 = jnp.maximum(
                jnp.dot(xs, W1_ref[0], preferred_element_type=jnp.float32), 0.0)
            o0 = jnp.dot(h0, W2_ref[0], preferred_element_type=jnp.float32) * w0
            h1 = jnp.maximum(
                jnp.dot(xs, W1_ref[1], preferred_element_type=jnp.float32), 0.0)
            o1 = jnp.dot(h1, W2_ref[1], preferred_element_type=jnp.float32) * w1
            pbuf_ref[s] = o0 + o1

        rdma_p = pltpu.make_async_remote_copy(
            src_ref=pbuf_ref.at[other_z], dst_ref=cbuf_ref,
            send_sem=send_sems.at[2], recv_sem=recv_sems.at[2],
            device_id=partner, device_id_type=pl.DeviceIdType.MESH)
        rdma_p.start()
        rdma_p.wait()
        out_ref[...] = pbuf_ref[my_z] + cbuf_ref[...]

    out_shape = jax.ShapeDtypeStruct((t_per, d), jnp.float32)
    return pl.pallas_call(
        body,
        out_shape=out_shape,
        in_specs=[pl.BlockSpec(memory_space=pltpu.VMEM)] * 4,
        out_specs=pl.BlockSpec(memory_space=pltpu.VMEM),
        scratch_shapes=[
            pltpu.VMEM((N_Z, t_per, d), jnp.float32),
            pltpu.VMEM((N_Z, e_per, d), jnp.float32),
            pltpu.VMEM((N_Z, t_per, d), jnp.float32),
            pltpu.VMEM((t_per, d), jnp.float32),
            pltpu.SemaphoreType.DMA((3,)),
            pltpu.SemaphoreType.DMA((3,)),
        ],
        compiler_params=pltpu.CompilerParams(collective_id=0),
    )(x, router_t, W1, W2)


# device time: 25832 ns/iter; 1.0778x vs baseline; 1.0778x over previous
import jax
import jax.numpy as jnp
from jax import lax
from jax.experimental import pallas as pl
from jax.experimental.pallas import tpu as pltpu

N_CHUNK = 2

_GATE_DIMS = (((1,), (1,)), ((), ()))


def kernel(x, router, W1, W2):
    t_per, d = x.shape
    e_per = W1.shape[0]
    router_t = router.T
    rows = t_per // N_CHUNK

    def body(x_ref, rt_ref, W1_ref, W2_ref, out_ref,
             xrecv_ref, rrecv_ref, psend_ref, cbuf_ref, send_sems, recv_sems):
        my_x = lax.axis_index("x")
        my_y = lax.axis_index("y")
        my_z = lax.axis_index("z")
        partner = (my_x, my_y, 1 - my_z)

        barrier_sem = pltpu.get_barrier_semaphore()
        pl.semaphore_signal(barrier_sem, inc=1, device_id=partner,
                            device_id_type=pl.DeviceIdType.MESH)
        pl.semaphore_wait(barrier_sem, 1)

        rdma_r = pltpu.make_async_remote_copy(
            src_ref=rt_ref, dst_ref=rrecv_ref,
            send_sem=send_sems.at[1], recv_sem=recv_sems.at[1],
            device_id=partner, device_id_type=pl.DeviceIdType.MESH)
        rdma_r.start()
        rdma_x = pltpu.make_async_remote_copy(
            src_ref=x_ref, dst_ref=xrecv_ref,
            send_sem=send_sems.at[0], recv_sem=recv_sems.at[0],
            device_id=partner, device_id_type=pl.DeviceIdType.MESH)
        rdma_x.start()

        def topk_weights(gm, go):
            a0, a1 = gm[:, 0:1], gm[:, 1:2]
            b0, b1 = go[:, 0:1], go[:, 1:2]
            ma, sa = jnp.maximum(a0, a1), jnp.minimum(a0, a1)
            mb, sb = jnp.maximum(b0, b1), jnp.minimum(b0, b1)
            m1 = jnp.maximum(ma, mb)
            m2 = jnp.where(ma >= mb, jnp.maximum(sa, mb), jnp.maximum(sb, ma))
            t1 = 1.0 / (1.0 + jnp.exp(m2 - m1))
            t2 = 1.0 - t1

            def wexp(g):
                return jnp.where(g == m1, t1, jnp.where(g == m2, t2, 0.0))

            return wexp(a0), wexp(a1)

        def ffn(xs):
            h0 = jnp.maximum(
                jnp.dot(xs, W1_ref[0], preferred_element_type=jnp.float32), 0.0)
            o0 = jnp.dot(h0, W2_ref[0], preferred_element_type=jnp.float32)
            h1 = jnp.maximum(
                jnp.dot(xs, W1_ref[1], preferred_element_type=jnp.float32), 0.0)
            o1 = jnp.dot(h1, W2_ref[1], preferred_element_type=jnp.float32)
            return o0, o1

        xs = x_ref[...]
        o0, o1 = ffn(xs)
        g_mine = lax.dot_general(xs, rt_ref[...], _GATE_DIMS,
                                 preferred_element_type=jnp.float32)
        rdma_r.wait()
        g_oth = lax.dot_general(xs, rrecv_ref[...], _GATE_DIMS,
                                preferred_element_type=jnp.float32)
        w0, w1 = topk_weights(g_mine, g_oth)
        out_ref[...] = o0 * w0 + o1 * w1

        rdma_x.wait()
        xp = xrecv_ref[...]
        gp_mine = lax.dot_general(xp, rt_ref[...], _GATE_DIMS,
                                  preferred_element_type=jnp.float32)
        gp_oth = lax.dot_general(xp, rrecv_ref[...], _GATE_DIMS,
                                 preferred_element_type=jnp.float32)
        wp0, wp1 = topk_weights(gp_mine, gp_oth)
        rdma_p = []
        for c in range(N_CHUNK):
            lo = c * rows
            oc0, oc1 = ffn(xp[lo:lo + rows])
            psend_ref[c] = (oc0 * wp0[lo:lo + rows] + oc1 * wp1[lo:lo + rows])
            r = pltpu.make_async_remote_copy(
                src_ref=psend_ref.at[c], dst_ref=cbuf_ref.at[c],
                send_sem=send_sems.at[2 + c], recv_sem=recv_sems.at[2 + c],
                device_id=partner, device_id_type=pl.DeviceIdType.MESH)
            r.start()
            rdma_p.append(r)

        for c in range(N_CHUNK):
            rdma_p[c].wait()
            lo = c * rows
            out_ref[lo:lo + rows, :] = out_ref[lo:lo + rows, :] + cbuf_ref[c]

    out_shape = jax.ShapeDtypeStruct((t_per, d), jnp.float32)
    return pl.pallas_call(
        body,
        out_shape=out_shape,
        in_specs=[pl.BlockSpec(memory_space=pltpu.VMEM)] * 4,
        out_specs=pl.BlockSpec(memory_space=pltpu.VMEM),
        scratch_shapes=[
            pltpu.VMEM((t_per, d), jnp.float32),
            pltpu.VMEM((e_per, d), jnp.float32),
            pltpu.VMEM((N_CHUNK, rows, d), jnp.float32),
            pltpu.VMEM((N_CHUNK, rows, d), jnp.float32),
            pltpu.SemaphoreType.DMA((2 + N_CHUNK,)),
            pltpu.SemaphoreType.DMA((2 + N_CHUNK,)),
        ],
        compiler_params=pltpu.CompilerParams(collective_id=0),
    )(x, router_t, W1, W2)
